# baseline (speedup 1.0000x reference)
import functools

import jax
import jax.numpy as jnp
from jax import lax
from jax.experimental import pallas as pl
from jax.experimental.pallas import tpu as pltpu
from jax.experimental.pallas import tpu_sc as plsc

DIM = 64
BATCH = 16384
LANES = 16
IDX_CHUNK = 128
PAIR_ROWS = 500000


def _make_kernel(num_cores, num_subcores):
    nw = num_cores * num_subcores
    b_per_w = BATCH // nw
    n_chunks = b_per_w // IDX_CHUNK
    mesh = plsc.VectorSubcoreMesh(core_axis_name="c", subcore_axis_name="s")

    @functools.partial(
        pl.kernel,
        mesh=mesh,
        compiler_params=pltpu.CompilerParams(use_tc_tiling_on_sc=True),
        out_type=jax.ShapeDtypeStruct((BATCH,), jnp.float32),
        scratch_types=[
            pltpu.VMEM((n_chunks, IDX_CHUNK), jnp.int32),
            pltpu.VMEM((n_chunks, IDX_CHUNK), jnp.int32),
            pltpu.VMEM((n_chunks, IDX_CHUNK), jnp.int32),
            pltpu.VMEM((n_chunks, IDX_CHUNK), jnp.int32),
            pltpu.VMEM((2, IDX_CHUNK, 2 * DIM), jnp.float32),
            pltpu.VMEM((2, IDX_CHUNK, 2 * DIM), jnp.float32),
            pltpu.VMEM((b_per_w,), jnp.float32),
            pltpu.SemaphoreType.DMA,
            pltpu.SemaphoreType.DMA,
            pltpu.SemaphoreType.DMA,
            pltpu.SemaphoreType.DMA,
        ],
    )
    def two_tower(uid_hbm, iid_hbm, uemb_hbm, iemb_hbm, out_hbm,
                  uid_v, iid_v, ugidx, igidx, urows, vrows, out_v,
                  su0, su1, sv0, sv1):
        wid = lax.axis_index("s") * num_cores + lax.axis_index("c")
        base = wid * b_per_w
        chunk0 = wid * n_chunks
        pltpu.sync_copy(uid_hbm.at[pl.ds(chunk0, n_chunks)], uid_v)
        pltpu.sync_copy(iid_hbm.at[pl.ds(chunk0, n_chunks)], iid_v)

        for j in range(n_chunks):
            for g in range(IDX_CHUNK // LANES):
                sl = pl.ds(g * LANES, LANES)
                ugidx[j, sl] = jnp.right_shift(uid_v[j, sl], 1)
                igidx[j, sl] = jnp.right_shift(iid_v[j, sl], 1)

        usems = [su0, su1]
        vsems = [sv0, sv1]

        def fire(j):
            bi = j % 2
            cu = pltpu.make_async_copy(uemb_hbm.at[ugidx.at[j]], urows.at[bi],
                                       usems[bi])
            cv = pltpu.make_async_copy(iemb_hbm.at[igidx.at[j]], vrows.at[bi],
                                       vsems[bi])
            cu.start()
            cv.start()
            return cu, cv

        lane = lax.iota(jnp.int32, LANES)
        pend = {}
        pend[0] = fire(0)
        pend[1] = fire(1)
        for j in range(n_chunks):
            bi = j % 2
            cu, cv = pend.pop(j)
            cu.wait()
            cv.wait()

            def body(g, carry, j=j, bi=bi):
                uvec = uid_v[j, pl.ds(g * LANES, LANES)]
                ivec = iid_v[j, pl.ds(g * LANES, LANES)]
                res = jnp.zeros((LANES,), jnp.float32)
                for r in range(LANES):
                    b = g * LANES + r
                    uh = jnp.full((LANES,), (uvec[r] & 1).astype(jnp.float32))
                    ih = jnp.full((LANES,), (ivec[r] & 1).astype(jnp.float32))
                    acc = None
                    for c in range(DIM // LANES):
                        ua = urows[bi, b, pl.ds(c * LANES, LANES)]
                        ub = urows[bi, b, pl.ds(DIM + c * LANES, LANES)]
                        va = vrows[bi, b, pl.ds(c * LANES, LANES)]
                        vb = vrows[bi, b, pl.ds(DIM + c * LANES, LANES)]
                        u = ua + (ub - ua) * uh
                        v = va + (vb - va) * ih
                        acc = u * v if acc is None else acc + u * v
                    folded = acc + lax.rev(acc, (0,))
                    s = folded[0]
                    for k in range(1, LANES // 2):
                        s = s + folded[k]
                    res = jnp.where(lane == r, s, res)
                out_v[pl.ds((j * IDX_CHUNK // LANES + g) * LANES, LANES)] = res
                return carry

            lax.fori_loop(0, IDX_CHUNK // LANES, body, 0)
            if j + 2 < n_chunks:
                pend[j + 2] = fire(j + 2)

        pltpu.sync_copy(out_v, out_hbm.at[pl.ds(base, b_per_w)])

    return two_tower


@jax.jit
def kernel(user_ids, item_ids, user_emb, item_emb):
    info = plsc.get_sparse_core_info()
    k = _make_kernel(info.num_cores, info.num_subcores)
    uid = user_ids.reshape(BATCH // IDX_CHUNK, IDX_CHUNK)
    iid = item_ids.reshape(BATCH // IDX_CHUNK, IDX_CHUNK)
    uemb = user_emb.reshape(PAIR_ROWS, 2 * DIM)
    iemb = item_emb.reshape(PAIR_ROWS, 2 * DIM)
    return k(uid, iid, uemb, iemb)
